# baseline (device time: 1674394 ns/iter reference)
import jax
import jax.numpy as jnp
from jax import lax
from jax.experimental import pallas as pl
from jax.experimental.pallas import tpu as pltpu

N_DEV = 4
SCALE = 0.08838834764831843
S_LOC = 2048
D = 1024
HQ = 8
DH = 128
N_CLS = 4
S_CLS = S_LOC // N_CLS
BLK = 64
N_BLK = S_CLS // BLK

_MESH = pl.DeviceIdType.MESH

_CHUNKS = (
    [("loc", c) for c in range(N_CLS)]
    + [("left", c) for c in range(N_CLS)]
    + [("right", c) for c in range(N_CLS)]
    + [("diag", c) for c in (0, 2, 1, 3)]
)


def _class_unsort(a):
    f = a.shape[-1]
    return a.reshape(4, 8, 64, f).transpose(1, 0, 2, 3).reshape(S_LOC, f)


def _fused_body(
    x_ref, wq_ref, k_ref, v_ref, o_ref, kvall, raw_stage,
    q_cls, buf, l_scr,
    dma_sems, sort_sems, p1_send, p1_recv, p2_send, p2_recv,
):
    my = lax.axis_index("i")
    left = lax.rem(my + N_DEV - 1, N_DEV)
    right = lax.rem(my + 1, N_DEV)
    diag = lax.rem(my + 2, N_DEV)

    bar = pltpu.get_barrier_semaphore()
    for nbr in (left, right):
        pl.semaphore_signal(bar, inc=1, device_id=(nbr,), device_id_type=_MESH)
    pl.semaphore_wait(bar, 2)

    def remote(src, dst, ssem, rsem, dev):
        return pltpu.make_async_remote_copy(
            src_ref=src, dst_ref=dst, send_sem=ssem, recv_sem=rsem,
            device_id=(dev,), device_id_type=_MESH,
        )

    p1 = [
        remote(k_ref, raw_stage.at[1, :, 0:D], p1_send.at[0],
               p1_recv.at[0], left),
        remote(v_ref, raw_stage.at[1, :, D:2 * D], p1_send.at[1],
               p1_recv.at[1], left),
        remote(k_ref, raw_stage.at[0, :, 0:D], p1_send.at[2],
               p1_recv.at[2], right),
        remote(v_ref, raw_stage.at[0, :, D:2 * D], p1_send.at[3],
               p1_recv.at[3], right),
    ]
    for s in p1:
        s.start()

    r_left_k = remote(k_ref, raw_stage.at[0, :, 0:D],
                      p1_send.at[2], p1_recv.at[2], left)
    r_left_v = remote(v_ref, raw_stage.at[0, :, D:2 * D],
                      p1_send.at[3], p1_recv.at[3], left)
    r_right_k = remote(k_ref, raw_stage.at[1, :, 0:D],
                       p1_send.at[0], p1_recv.at[0], right)
    r_right_v = remote(v_ref, raw_stage.at[1, :, D:2 * D],
                       p1_send.at[1], p1_recv.at[1], right)

    fwd = {
        0: remote(kvall.at[left, 0, 0], kvall.at[left, 0, 0],
                  p2_send.at[0], p2_recv.at[0], right),
        1: remote(kvall.at[left, 0, 1], kvall.at[left, 0, 1],
                  p2_send.at[1], p2_recv.at[1], right),
        2: remote(kvall.at[right, 1, 0], kvall.at[right, 1, 0],
                  p2_send.at[2], p2_recv.at[2], left),
        3: remote(kvall.at[right, 1, 1], kvall.at[right, 1, 1],
                  p2_send.at[3], p2_recv.at[3], left),
    }
    r_diag = {
        0: remote(kvall.at[diag, 0, 0], kvall.at[diag, 0, 0],
                  p2_send.at[0], p2_recv.at[0], left),
        1: remote(kvall.at[diag, 0, 1], kvall.at[diag, 0, 1],
                  p2_send.at[1], p2_recv.at[1], left),
        2: remote(kvall.at[diag, 1, 0], kvall.at[diag, 1, 0],
                  p2_send.at[2], p2_recv.at[2], right),
        3: remote(kvall.at[diag, 1, 1], kvall.at[diag, 1, 1],
                  p2_send.at[3], p2_recv.at[3], right),
    }

    sort_descs = {}

    def issue_sort(side_idx, dev):
        for c in range(N_CLS):
            sem = sort_sems.at[side_idx, c]
            descs = []
            for m in range(N_BLK):
                r = c * BLK + m * (N_CLS * BLK)
                dst_rows = slice(m * BLK, (m + 1) * BLK)
                dst = kvall.at[dev, c // 2, c % 2, dst_rows, :]
                if side_idx == 0:
                    descs.append(pltpu.make_async_copy(
                        k_ref.at[r:r + BLK, :],
                        kvall.at[dev, c // 2, c % 2, dst_rows, 0:D], sem))
                    descs.append(pltpu.make_async_copy(
                        v_ref.at[r:r + BLK, :],
                        kvall.at[dev, c // 2, c % 2, dst_rows, D:2 * D], sem))
                else:
                    descs.append(pltpu.make_async_copy(
                        raw_stage.at[side_idx - 1, r:r + BLK, :], dst, sem))
            for d in descs:
                d.start()
            sort_descs[(side_idx, c)] = descs

    _sorted = set()

    def ensure_sorted(side_idx, c):
        if (side_idx, c) in _sorted:
            return
        for d in sort_descs[(side_idx, c)]:
            d.wait()
        _sorted.add((side_idx, c))

    issue_sort(0, my)

    wq = wq_ref[...]
    for c in range(N_CLS):
        for m in range(N_BLK):
            r = c * BLK + m * (N_CLS * BLK)
            q_cls[c, m * BLK:(m + 1) * BLK, :] = jnp.dot(
                x_ref[r:r + BLK, :], wq, preferred_element_type=jnp.float32)

    o_ref[...] = jnp.zeros_like(o_ref)
    l_scr[...] = jnp.zeros_like(l_scr)

    side_of = {"loc": 0, "left": 1, "right": 2}

    def start_chunk(i):
        kind, c = _CHUNKS[i]
        slot = i % 2
        if kind != "diag":
            ensure_sorted(side_of[kind], c)
        dev = {"loc": my, "left": left, "right": right, "diag": diag}[kind]
        cp = pltpu.make_async_copy(
            kvall.at[dev, c // 2, c % 2], buf.at[slot], dma_sems.at[slot])
        cp.start()
        return cp

    def gate(i):
        if i == 4:
            r_left_k.wait_recv()
            r_left_v.wait_recv()
            issue_sort(1, left)
            ensure_sorted(1, 0)
            ensure_sorted(1, 1)
            fwd[0].start()
            fwd[1].start()
        elif i == 8:
            r_right_k.wait_recv()
            r_right_v.wait_recv()
            issue_sort(2, right)
            ensure_sorted(2, 2)
            ensure_sorted(2, 3)
            fwd[2].start()
            fwd[3].start()
        elif i >= 12:
            r_diag[_CHUNKS[i][1]].wait_recv()

    def compute_chunk(i):
        _, c = _CHUNKS[i]
        kv = buf[i % 2]
        q = q_cls[c]
        for h in range(HQ):
            qh = q[:, h * DH:(h + 1) * DH]
            kh = kv[:, h * DH:(h + 1) * DH]
            vh = kv[:, D + h * DH:D + (h + 1) * DH]
            s = lax.dot_general(
                qh, kh, (((1,), (1,)), ((), ())),
                preferred_element_type=jnp.float32,
            ) * SCALE
            p = jnp.exp(s)
            lsum = jnp.sum(p, axis=1, keepdims=True)
            l_scr[c, h] += jnp.broadcast_to(lsum, (S_CLS, DH))
            o_ref[c, :, h * DH:(h + 1) * DH] += lax.dot_general(
                p, vh, (((1,), (0,)), ((), ())),
                preferred_element_type=jnp.float32,
            )

    n = len(_CHUNKS)
    chunks = {0: start_chunk(0)}
    for i in range(n):
        if i + 1 < n:
            gate(i + 1)
            chunks[i + 1] = start_chunk(i + 1)
        chunks[i].wait()
        compute_chunk(i)
        if i >= 12:
            c = _CHUNKS[i][1]
            for h in range(HQ):
                o_ref[c, :, h * DH:(h + 1) * DH] = (
                    o_ref[c, :, h * DH:(h + 1) * DH] / l_scr[c, h]
                )

    for s in p1:
        s.wait_send()
    for c in range(N_CLS):
        fwd[c].wait_send()


def kernel(x, Wq, K_ext, V_ext, Wo):
    x2 = x[0]
    k2 = K_ext[0].reshape(S_LOC, D)
    v2 = V_ext[0].reshape(S_LOC, D)

    ctx, _, _ = pl.pallas_call(
        _fused_body,
        out_shape=(
            jax.ShapeDtypeStruct((N_CLS, S_CLS, D), jnp.float32),
            jax.ShapeDtypeStruct((N_DEV, 2, 2, S_CLS, 2 * D), jnp.float32),
            jax.ShapeDtypeStruct((2, S_LOC, 2 * D), jnp.float32),
        ),
        in_specs=[
            pl.BlockSpec(memory_space=pltpu.MemorySpace.VMEM),
            pl.BlockSpec(memory_space=pltpu.MemorySpace.VMEM),
            pl.BlockSpec(memory_space=pl.ANY),
            pl.BlockSpec(memory_space=pl.ANY),
        ],
        out_specs=(
            pl.BlockSpec(memory_space=pltpu.MemorySpace.VMEM),
            pl.BlockSpec(memory_space=pl.ANY),
            pl.BlockSpec(memory_space=pl.ANY),
        ),
        scratch_shapes=[
            pltpu.VMEM((N_CLS, S_CLS, D), jnp.float32),
            pltpu.VMEM((2, S_CLS, 2 * D), jnp.float32),
            pltpu.VMEM((N_CLS, HQ, S_CLS, DH), jnp.float32),
            pltpu.SemaphoreType.DMA((2,)),
            pltpu.SemaphoreType.DMA((3, N_CLS)),
            pltpu.SemaphoreType.DMA((4,)),
            pltpu.SemaphoreType.DMA((4,)),
            pltpu.SemaphoreType.DMA((4,)),
            pltpu.SemaphoreType.DMA((4,)),
        ],
        compiler_params=pltpu.CompilerParams(
            collective_id=0,
            vmem_limit_bytes=64 * 1024 * 1024,
        ),
    )(x2, Wq, k2, v2)

    out = _class_unsort(ctx) @ Wo
    return out[None]


# device time: 334504 ns/iter; 5.0056x vs baseline; 5.0056x over previous
import jax
import jax.numpy as jnp
from jax import lax
from jax.experimental import pallas as pl
from jax.experimental.pallas import tpu as pltpu

N_DEV = 4
SCALE = 0.08838834764831843
S_LOC = 2048
D = 1024
HQ = 8
DH = 128
N_CLS = 4
S_CLS = S_LOC // N_CLS

_MESH = pl.DeviceIdType.MESH

_CHUNKS = (
    [("loc", c) for c in range(N_CLS)]
    + [("left", c) for c in range(N_CLS)]
    + [("right", c) for c in range(N_CLS)]
    + [("diag", c) for c in (0, 2, 1, 3)]
)


def _class_sort(a):
    f = a.shape[-1]
    return (
        a.reshape(8, 4, 64, f).transpose(1, 0, 2, 3).reshape(N_CLS, S_CLS, f)
    )


def _class_unsort(a):
    f = a.shape[-1]
    return a.reshape(4, 8, 64, f).transpose(1, 0, 2, 3).reshape(S_LOC, f)


def _fused_body(
    xs_ref, wq_ref, k_ref, v_ref, o_ref, kvall,
    q_cls, buf, l_scr,
    dma_sems, p1_send, p1_recv, p2_send, p2_recv,
):
    my = lax.axis_index("i")
    left = lax.rem(my + N_DEV - 1, N_DEV)
    right = lax.rem(my + 1, N_DEV)
    diag = lax.rem(my + 2, N_DEV)

    bar = pltpu.get_barrier_semaphore()
    for nbr in (left, right):
        pl.semaphore_signal(bar, inc=1, device_id=(nbr,), device_id_type=_MESH)
    pl.semaphore_wait(bar, 2)

    def remote(src, dst, ssem, rsem, dev):
        return pltpu.make_async_remote_copy(
            src_ref=src, dst_ref=dst, send_sem=ssem, recv_sem=rsem,
            device_id=(dev,), device_id_type=_MESH,
        )

    p1 = [
        remote(k_ref, kvall.at[my, :, :, :, 0:D], p1_send.at[0],
               p1_recv.at[0], left),
        remote(v_ref, kvall.at[my, :, :, :, D:2 * D], p1_send.at[1],
               p1_recv.at[1], left),
        remote(k_ref, kvall.at[my, :, :, :, 0:D], p1_send.at[2],
               p1_recv.at[2], right),
        remote(v_ref, kvall.at[my, :, :, :, D:2 * D], p1_send.at[3],
               p1_recv.at[3], right),
    ]
    for s in p1:
        s.start()

    r_left_k = remote(k_ref, kvall.at[left, :, :, :, 0:D],
                      p1_send.at[2], p1_recv.at[2], left)
    r_left_v = remote(v_ref, kvall.at[left, :, :, :, D:2 * D],
                      p1_send.at[3], p1_recv.at[3], left)
    r_right_k = remote(k_ref, kvall.at[right, :, :, :, 0:D],
                       p1_send.at[0], p1_recv.at[0], right)
    r_right_v = remote(v_ref, kvall.at[right, :, :, :, D:2 * D],
                       p1_send.at[1], p1_recv.at[1], right)

    fwd = {
        0: remote(kvall.at[left, 0, 0], kvall.at[left, 0, 0],
                  p2_send.at[0], p2_recv.at[0], right),
        1: remote(kvall.at[left, 0, 1], kvall.at[left, 0, 1],
                  p2_send.at[1], p2_recv.at[1], right),
        2: remote(kvall.at[right, 1, 0], kvall.at[right, 1, 0],
                  p2_send.at[2], p2_recv.at[2], left),
        3: remote(kvall.at[right, 1, 1], kvall.at[right, 1, 1],
                  p2_send.at[3], p2_recv.at[3], left),
    }
    r_diag = {
        0: remote(kvall.at[diag, 0, 0], kvall.at[diag, 0, 0],
                  p2_send.at[0], p2_recv.at[0], left),
        1: remote(kvall.at[diag, 0, 1], kvall.at[diag, 0, 1],
                  p2_send.at[1], p2_recv.at[1], left),
        2: remote(kvall.at[diag, 1, 0], kvall.at[diag, 1, 0],
                  p2_send.at[2], p2_recv.at[2], right),
        3: remote(kvall.at[diag, 1, 1], kvall.at[diag, 1, 1],
                  p2_send.at[3], p2_recv.at[3], right),
    }

    wq = wq_ref[...]
    for c in range(N_CLS):
        q_cls[c] = jnp.dot(xs_ref[c], wq, preferred_element_type=jnp.float32)

    o_ref[...] = jnp.zeros_like(o_ref)
    l_scr[...] = jnp.zeros_like(l_scr)

    def start_chunk(i):
        src_kind, c = _CHUNKS[i]
        slot = i % 2
        if src_kind == "loc":
            cps = [
                pltpu.make_async_copy(
                    k_ref.at[c // 2, c % 2], buf.at[slot, :, 0:D],
                    dma_sems.at[slot, 0]),
                pltpu.make_async_copy(
                    v_ref.at[c // 2, c % 2], buf.at[slot, :, D:2 * D],
                    dma_sems.at[slot, 1]),
            ]
        else:
            s = {"left": left, "right": right, "diag": diag}[src_kind]
            cps = [pltpu.make_async_copy(
                kvall.at[s, c // 2, c % 2], buf.at[slot],
                dma_sems.at[slot, 0])]
        for cp in cps:
            cp.start()
        return cps

    def gate(i):
        if i == 4:
            r_left_k.wait_recv()
            r_left_v.wait_recv()
            fwd[0].start()
            fwd[1].start()
        elif i == 8:
            r_right_k.wait_recv()
            r_right_v.wait_recv()
            fwd[2].start()
            fwd[3].start()
        elif i >= 12:
            r_diag[_CHUNKS[i][1]].wait_recv()

    def compute_chunk(i):
        _, c = _CHUNKS[i]
        kv = buf[i % 2]
        q = q_cls[c]
        for h in range(HQ):
            qh = q[:, h * DH:(h + 1) * DH]
            kh = kv[:, h * DH:(h + 1) * DH]
            vh = kv[:, D + h * DH:D + (h + 1) * DH]
            s = lax.dot_general(
                qh, kh, (((1,), (1,)), ((), ())),
                preferred_element_type=jnp.float32,
            ) * SCALE
            p = jnp.exp(s)
            lsum = jnp.sum(p, axis=1, keepdims=True)
            l_scr[c, h] += jnp.broadcast_to(lsum, (S_CLS, DH))
            o_ref[c, :, h * DH:(h + 1) * DH] += lax.dot_general(
                p, vh, (((1,), (0,)), ((), ())),
                preferred_element_type=jnp.float32,
            )

    n = len(_CHUNKS)
    chunks = {0: start_chunk(0)}
    for i in range(n):
        if i + 1 < n:
            gate(i + 1)
            chunks[i + 1] = start_chunk(i + 1)
        for cp in chunks[i]:
            cp.wait()
        compute_chunk(i)
        if i >= 12:
            c = _CHUNKS[i][1]
            for h in range(HQ):
                o_ref[c, :, h * DH:(h + 1) * DH] = (
                    o_ref[c, :, h * DH:(h + 1) * DH] / l_scr[c, h]
                )

    for s in p1:
        s.wait_send()
    for c in range(N_CLS):
        fwd[c].wait_send()


def kernel(x, Wq, K_ext, V_ext, Wo):
    xs = _class_sort(x[0])
    ks = _class_sort(K_ext[0].reshape(S_LOC, D)).reshape(2, 2, S_CLS, D)
    vs = _class_sort(V_ext[0].reshape(S_LOC, D)).reshape(2, 2, S_CLS, D)

    ctx, _ = pl.pallas_call(
        _fused_body,
        out_shape=(
            jax.ShapeDtypeStruct((N_CLS, S_CLS, D), jnp.float32),
            jax.ShapeDtypeStruct((N_DEV, 2, 2, S_CLS, 2 * D), jnp.float32),
        ),
        in_specs=[
            pl.BlockSpec(memory_space=pltpu.MemorySpace.VMEM),
            pl.BlockSpec(memory_space=pltpu.MemorySpace.VMEM),
            pl.BlockSpec(memory_space=pl.ANY),
            pl.BlockSpec(memory_space=pl.ANY),
        ],
        out_specs=(
            pl.BlockSpec(memory_space=pltpu.MemorySpace.VMEM),
            pl.BlockSpec(memory_space=pl.ANY),
        ),
        scratch_shapes=[
            pltpu.VMEM((N_CLS, S_CLS, D), jnp.float32),
            pltpu.VMEM((2, S_CLS, 2 * D), jnp.float32),
            pltpu.VMEM((N_CLS, HQ, S_CLS, DH), jnp.float32),
            pltpu.SemaphoreType.DMA((2, 2)),
            pltpu.SemaphoreType.DMA((4,)),
            pltpu.SemaphoreType.DMA((4,)),
            pltpu.SemaphoreType.DMA((4,)),
            pltpu.SemaphoreType.DMA((4,)),
        ],
        compiler_params=pltpu.CompilerParams(
            collective_id=0,
            vmem_limit_bytes=64 * 1024 * 1024,
        ),
    )(xs, Wq, ks, vs)

    out = _class_unsort(ctx) @ Wo
    return out[None]
